# baseline (speedup 1.0000x reference)
import functools

import jax
import jax.numpy as jnp
from jax import lax
from jax.experimental import pallas as pl
from jax.experimental.pallas import tpu as pltpu
from jax.experimental.pallas import tpu_sc as plsc

B = 1024
L = 50
C = 1000
LP = 64
NW = 32
CROWS = 40
CPP = C // CROWS
NCHUNK = L * CPP
NC_LO = NCHUNK // NW
NC_REM = NCHUNK - NC_LO * NW
NC_MAX = NC_LO + 1
NBUF = 3


def _onehot_body(t_hbm, z_hbm, y_hbm, t_v, buf0, buf1, buf2,
                 sem0, sem1, sem2):
    wid = lax.axis_index("s") * 2 + lax.axis_index("c")
    nc = jnp.where(wid < NC_REM, NC_LO + 1, NC_LO)
    qc0 = wid * NC_LO + lax.min(wid, NC_REM)

    lfirst = qc0 // CPP
    hs = [pltpu.async_copy(t_hbm.at[lfirst + j], t_v.at[j], sem0)
          for j in range(4)]
    hz = [pltpu.async_copy(z_hbm, b, sem1) for b in (buf0, buf1, buf2)]
    for h in hs + hz:
        h.wait()

    zeros16 = jnp.zeros((16,), jnp.float32)
    ones16 = jnp.ones((16,), jnp.float32)
    iota16 = lax.iota(jnp.int32, 16)
    bufs = (buf0, buf1, buf2)
    sems = (sem0, sem1, sem2)

    def _scatter(buf, qc, val16):
        l = qc // CPP
        c_lo = (qc - l * CPP) * CROWS

        def _v(v4, carry):
            for u in range(4):
                v = v4 * 4 + u
                tl = t_v[l - lfirst, pl.ds(v * 16, 16)]
                m = (tl >= c_lo) & (tl < c_lo + CROWS)
                plsc.store_scatter(buf, [tl - c_lo, iota16 + v * 16], val16,
                                   mask=m)
            return carry

        lax.fori_loop(0, B // 64, _v, 0)
        return l, c_lo

    def _round(k3, carry):
        for part in range(NBUF):
            k = k3 * NBUF + part
            buf, sem = bufs[part], sems[part]

            @pl.when(k < nc)
            def _():
                @pl.when(k >= NBUF)
                def _():
                    pltpu.make_async_copy(
                        buf, y_hbm.at[0, pl.ds(0, CROWS)], sem).wait()
                    _scatter(buf, qc0 + k - NBUF, zeros16)

                l, c_lo = _scatter(buf, qc0 + k, ones16)
                pltpu.async_copy(buf, y_hbm.at[l, pl.ds(c_lo, CROWS)], sem)

        return carry

    lax.fori_loop(0, (NC_MAX + NBUF - 1) // NBUF, _round, 0)

    for buf, sem in zip(bufs, sems):
        pltpu.make_async_copy(buf, y_hbm.at[0, pl.ds(0, CROWS)], sem).wait()


_onehot_sc = functools.partial(
    pl.kernel,
    mesh=plsc.VectorSubcoreMesh(core_axis_name="c", subcore_axis_name="s"),
    out_type=jax.ShapeDtypeStruct((L, C, B), jnp.float32),
    scratch_types=[
        pltpu.VMEM((4, B), jnp.int32),
        pltpu.VMEM((CROWS, B), jnp.float32),
        pltpu.VMEM((CROWS, B), jnp.float32),
        pltpu.VMEM((CROWS, B), jnp.float32),
        pltpu.SemaphoreType.DMA,
        pltpu.SemaphoreType.DMA,
        pltpu.SemaphoreType.DMA,
    ],
    compiler_params=pltpu.CompilerParams(needs_layout_passes=False),
)(_onehot_body)


def kernel(t, ones):
    del ones
    t_pad = jnp.pad(jnp.transpose(t.astype(jnp.int32)), ((0, LP - L), (0, 0)))
    z = jnp.zeros((CROWS, B), jnp.float32)
    return jnp.transpose(_onehot_sc(t_pad, z), (2, 1, 0))


# speedup vs baseline: 1.0365x; 1.0365x over previous
import functools

import jax
import jax.numpy as jnp
from jax import lax
from jax.experimental import pallas as pl
from jax.experimental.pallas import tpu as pltpu
from jax.experimental.pallas import tpu_sc as plsc

B = 1024
L = 50
C = 1000
LP = 64
NW = 32
ROWS = L * C
CROWS = 56
NFULL = ROWS // CROWS
TROWS = ROWS - NFULL * CROWS
NC_LO = NFULL // NW
NC_REM = NFULL - NC_LO * NW
NC_MAX = NC_LO + 1


def _onehot_body(t_hbm, z_hbm, y_hbm, t_v, buf0, buf1, sem0, sem1):
    wid = lax.axis_index("s") * 2 + lax.axis_index("c")
    nc = jnp.where(wid < NC_REM, NC_LO + 1, NC_LO)
    qc0 = wid * NC_LO + lax.min(wid, NC_REM)

    lfirst = (qc0 * CROWS) // C
    hs = [pltpu.async_copy(t_hbm.at[lfirst + j], t_v.at[j], sem0)
          for j in range(4)]
    hz = [pltpu.async_copy(z_hbm, b, sem1) for b in (buf0, buf1)]
    for h in hs + hz:
        h.wait()

    zeros16 = jnp.zeros((16,), jnp.float32)
    ones16 = jnp.ones((16,), jnp.float32)
    iota16 = lax.iota(jnp.int32, 16)
    bufs = (buf0, buf1)
    sems = (sem0, sem1)

    def _scan_plane(buf, l, c_lo, c_hi, row_ofs, val16):
        def _v(v4, carry):
            for u in range(4):
                v = v4 * 4 + u
                tl = t_v[l - lfirst, pl.ds(v * 16, 16)]
                m = (tl >= c_lo) & (tl < c_hi)
                plsc.store_scatter(
                    buf, [tl + (row_ofs - c_lo), iota16 + v * 16], val16,
                    mask=m)
            return carry

        lax.fori_loop(0, B // 64, _v, 0)

    def _scatter(buf, qc, val16):
        r0 = qc * CROWS
        la = r0 // C
        offa = r0 - la * C
        na = lax.min(C - offa, CROWS)
        _scan_plane(buf, la, offa, offa + na, 0, val16)

        @pl.when(na < CROWS)
        def _():
            _scan_plane(buf, la + 1, 0, CROWS - na, na, val16)

        return r0

    def _pair(k2, carry):
        for half in range(2):
            k = k2 * 2 + half
            buf, sem = bufs[half], sems[half]

            @pl.when(k < nc)
            def _():
                @pl.when(k >= 2)
                def _():
                    pltpu.make_async_copy(
                        buf, y_hbm.at[pl.ds(0, CROWS)], sem).wait()
                    _scatter(buf, qc0 + k - 2, zeros16)

                r0 = _scatter(buf, qc0 + k, ones16)
                pltpu.async_copy(buf, y_hbm.at[pl.ds(r0, CROWS)], sem)

        return carry

    lax.fori_loop(0, (NC_MAX + 1) // 2, _pair, 0)

    pltpu.make_async_copy(buf0, y_hbm.at[pl.ds(0, CROWS)], sem0).wait()
    pltpu.make_async_copy(buf1, y_hbm.at[pl.ds(0, CROWS)], sem1).wait()

    @pl.when(wid == NW - 1)
    def _():
        _scatter(buf0, qc0 + nc - 1, zeros16)
        _scan_plane(buf0, L - 1, C - TROWS, C, 0, ones16)
        pltpu.sync_copy(buf0.at[pl.ds(0, TROWS)],
                        y_hbm.at[pl.ds(ROWS - TROWS, TROWS)])


_onehot_sc = functools.partial(
    pl.kernel,
    mesh=plsc.VectorSubcoreMesh(core_axis_name="c", subcore_axis_name="s"),
    out_type=jax.ShapeDtypeStruct((ROWS, B), jnp.float32),
    scratch_types=[
        pltpu.VMEM((4, B), jnp.int32),
        pltpu.VMEM((CROWS, B), jnp.float32),
        pltpu.VMEM((CROWS, B), jnp.float32),
        pltpu.SemaphoreType.DMA,
        pltpu.SemaphoreType.DMA,
    ],
    compiler_params=pltpu.CompilerParams(needs_layout_passes=False),
)(_onehot_body)


def kernel(t, ones):
    del ones
    t_pad = jnp.pad(jnp.transpose(t.astype(jnp.int32)), ((0, LP - L), (0, 0)))
    z = jnp.zeros((CROWS, B), jnp.float32)
    y = _onehot_sc(t_pad, z).reshape(L, C, B)
    return jnp.transpose(y, (2, 1, 0))


# speedup vs baseline: 1.0509x; 1.0138x over previous
import functools

import jax
import jax.numpy as jnp
from jax import lax
from jax.experimental import pallas as pl
from jax.experimental.pallas import tpu as pltpu
from jax.experimental.pallas import tpu_sc as plsc

B = 1024
L = 50
C = 1000
LP = 64
NW = 32
CROWS = 40
CPP = C // CROWS
NCHUNK = L * CPP
NC_LO = NCHUNK // NW
NC_REM = NCHUNK - NC_LO * NW
NC_MAX = NC_LO + 1
TW = 16


def _onehot_body(t_hbm, z_hbm, y_hbm, t_v, buf0, buf1, sem0, sem1):
    wid = lax.axis_index("s") * 2 + lax.axis_index("c")
    nc = jnp.where(wid < NC_REM, NC_LO + 1, NC_LO)
    qc0 = wid * NC_LO + lax.min(wid, NC_REM)

    lw0 = (qc0 // CPP) // 8 * 8
    h_t = pltpu.async_copy(t_hbm.at[pl.ds(lw0, TW)], t_v, sem0)
    h_z0 = pltpu.async_copy(z_hbm, buf0, sem1)
    h_z1 = pltpu.async_copy(z_hbm, buf1, sem1)
    h_t.wait()
    h_z0.wait()
    h_z1.wait()

    zeros16 = jnp.zeros((16,), jnp.float32)
    ones16 = jnp.ones((16,), jnp.float32)
    iota16 = lax.iota(jnp.int32, 16)
    bufs = (buf0, buf1)
    sems = (sem0, sem1)

    def _scatter(buf, qc, val16):
        l = qc // CPP
        c_lo = (qc - l * CPP) * CROWS

        def _v(v4, carry):
            for u in range(4):
                v = v4 * 4 + u
                tl = t_v[l - lw0, pl.ds(v * 16, 16)]
                m = (tl >= c_lo) & (tl < c_lo + CROWS)
                plsc.store_scatter(buf, [tl - c_lo, iota16 + v * 16], val16,
                                   mask=m)
            return carry

        lax.fori_loop(0, B // 64, _v, 0)
        return l, c_lo

    def _pair(k2, carry):
        for half in range(2):
            k = k2 * 2 + half
            buf, sem = bufs[half], sems[half]

            @pl.when(k < nc)
            def _():
                @pl.when(k >= 2)
                def _():
                    pltpu.make_async_copy(
                        buf, y_hbm.at[0, pl.ds(0, CROWS)], sem).wait()
                    _scatter(buf, qc0 + k - 2, zeros16)

                l, c_lo = _scatter(buf, qc0 + k, ones16)
                pltpu.async_copy(buf, y_hbm.at[l, pl.ds(c_lo, CROWS)], sem)

        return carry

    lax.fori_loop(0, (NC_MAX + 1) // 2, _pair, 0)

    pltpu.make_async_copy(buf0, y_hbm.at[0, pl.ds(0, CROWS)], sem0).wait()
    pltpu.make_async_copy(buf1, y_hbm.at[0, pl.ds(0, CROWS)], sem1).wait()


_onehot_sc = functools.partial(
    pl.kernel,
    mesh=plsc.VectorSubcoreMesh(core_axis_name="c", subcore_axis_name="s"),
    out_type=jax.ShapeDtypeStruct((L, C, B), jnp.float32),
    scratch_types=[
        pltpu.VMEM((TW, B), jnp.int32),
        pltpu.VMEM((CROWS, B), jnp.float32),
        pltpu.VMEM((CROWS, B), jnp.float32),
        pltpu.SemaphoreType.DMA,
        pltpu.SemaphoreType.DMA,
    ],
    compiler_params=pltpu.CompilerParams(needs_layout_passes=False),
)(_onehot_body)


def kernel(t, ones):
    del ones
    t_pad = jnp.pad(jnp.transpose(t.astype(jnp.int32)), ((0, LP - L), (0, 0)))
    z = jnp.zeros((CROWS, B), jnp.float32)
    return jnp.transpose(_onehot_sc(t_pad, z), (2, 1, 0))
